# baseline (speedup 1.0000x reference)
import jax
import jax.numpy as jnp
from jax import lax
from jax.experimental import pallas as pl
from jax.experimental.pallas import tpu as pltpu
from jax.experimental.pallas import tpu_sc as plsc

VOCAB = 100000
EMB = 128
HID = 1024
NCLS = 1000
BATCH = 4096
SEQ = 50

NC = 2
NS = 16
NW = NC * NS
SEQP = 56
RPC = 2
IDXC = RPC * SEQP
LANES = 16
EV = EMB // LANES

SC_ROWS = 640
TC_ROWS = BATCH - SC_ROWS
BPW = SC_ROWS // NW
NCHUNK = BPW // RPC

NCLS_PAD = 1024
BM = 512
BMG = 576


def _sc_body(idx_hbm, emb_hbm, out_hbm, idx_v, rows0, rows1, out_v, sem0, sem1):
    wid = lax.axis_index("s") * NC + lax.axis_index("c")
    pltpu.sync_copy(idx_hbm.at[pl.ds(wid * (NCHUNK * IDXC), NCHUNK * IDXC)], idx_v)

    bufs = (rows0, rows1)
    sems = (sem0, sem1)

    def fire(c, j):
        pltpu.async_copy(emb_hbm.at[idx_v.at[pl.ds(c * IDXC, IDXC)]], bufs[j], sems[j])

    for j in range(2):
        fire(j, j)

    def pair(p, carry):
        for j in range(2):
            c = 2 * p + j
            pltpu.make_async_copy(emb_hbm.at[idx_v.at[pl.ds(c * IDXC, IDXC)]], bufs[j], sems[j]).wait()
            rows_v = bufs[j]
            for r in range(RPC):
                accs = [rows_v[r * SEQP, pl.ds(e * LANES, LANES)]
                        for e in range(EV)]
                for s in range(1, SEQP):
                    row = r * SEQP + s
                    for e in range(EV):
                        accs[e] = accs[e] + rows_v[row, pl.ds(e * LANES, LANES)]
                base = (c * RPC + r) * EMB
                for e in range(EV):
                    out_v[pl.ds(base + e * LANES, LANES)] = accs[e]

            @pl.when(c + 2 < NCHUNK)
            def _():
                fire(c + 2, j)

        return carry

    lax.fori_loop(0, NCHUNK // 2, pair, 0)
    pltpu.sync_copy(out_v, out_hbm.at[pl.ds(wid * (BPW * EMB), BPW * EMB)])


def _sc_sums(idx_flat, emb):
    mesh = plsc.VectorSubcoreMesh(core_axis_name="c", subcore_axis_name="s")
    return pl.kernel(
        _sc_body,
        out_type=jax.ShapeDtypeStruct((SC_ROWS * EMB,), jnp.float32),
        mesh=mesh,
        scratch_types=[
            pltpu.VMEM((NCHUNK * IDXC,), jnp.int32),
            pltpu.VMEM((IDXC, EMB), jnp.float32),
            pltpu.VMEM((IDXC, EMB), jnp.float32),
            pltpu.VMEM((BPW * EMB,), jnp.float32),
            pltpu.SemaphoreType.DMA,
            pltpu.SemaphoreType.DMA,
        ],
    )(idx_flat, emb)


def _tc_gather_body(text_smem, table_ref, out_ref):
    def row(b, carry):
        accs = [table_ref[pl.ds(text_smem[b, s], 1), :] for s in range(4)]
        for s in range(4, SEQ):
            accs[s % 4] = accs[s % 4] + table_ref[pl.ds(text_smem[b, s], 1), :]
        out_ref[pl.ds(b, 1), :] = (accs[0] + accs[1]) + (accs[2] + accs[3])
        return carry

    lax.fori_loop(0, BMG, row, 0, unroll=32)


def _tc_gather(text_tc, emb):
    return pl.pallas_call(
        _tc_gather_body,
        grid=(TC_ROWS // BMG,),
        in_specs=[
            pl.BlockSpec((BMG, SEQ), lambda i: (i, 0),
                         memory_space=pltpu.SMEM),
            pl.BlockSpec((VOCAB, EMB), lambda i: (0, 0)),
        ],
        out_specs=pl.BlockSpec((BMG, EMB), lambda i: (i, 0)),
        out_shape=jax.ShapeDtypeStruct((TC_ROWS, EMB), jnp.float32),
    )(text_tc, emb)


def _mlp_body(pad_ref, text_ref, sums_ref, w1_ref, b1_ref, w2_ref, b2_ref,
              w3_ref, b3_ref, out_ref):
    cnt = jnp.sum((text_ref[...] != pad_ref[0]).astype(jnp.float32), axis=1,
                  keepdims=True)
    x = sums_ref[...] / cnt
    h = jnp.dot(x, w1_ref[...], preferred_element_type=jnp.float32)
    h = jnp.maximum(h + b1_ref[...], 0.0)
    h = jnp.dot(h, w2_ref[...], preferred_element_type=jnp.float32)
    h = jnp.maximum(h + b2_ref[...], 0.0)
    h = jnp.dot(h, w3_ref[...], preferred_element_type=jnp.float32)
    h = h + b3_ref[...]
    out_ref[...] = jnp.concatenate(
        [h, jnp.zeros((BM, NCLS_PAD - NCLS), jnp.float32)], axis=1)


def _mlp(pad, text, sums, W1, b1, W2, b2, W3p, b3p):
    rows = text.shape[0]
    return pl.pallas_call(
        _mlp_body,
        grid=(rows // BM,),
        in_specs=[
            pl.BlockSpec(memory_space=pltpu.SMEM),
            pl.BlockSpec((BM, SEQ), lambda i: (i, 0)),
            pl.BlockSpec((BM, EMB), lambda i: (i, 0)),
            pl.BlockSpec((EMB, HID), lambda i: (0, 0)),
            pl.BlockSpec((1, HID), lambda i: (0, 0)),
            pl.BlockSpec((HID, HID), lambda i: (0, 0)),
            pl.BlockSpec((1, HID), lambda i: (0, 0)),
            pl.BlockSpec((HID, NCLS), lambda i: (0, 0)),
            pl.BlockSpec((1, NCLS), lambda i: (0, 0)),
        ],
        out_specs=pl.BlockSpec((BM, NCLS_PAD), lambda i: (i, 0)),
        out_shape=jax.ShapeDtypeStruct((rows, NCLS_PAD), jnp.float32),
    )(pad, text, sums, W1, b1, W2, b2, W3p, b3p)


def kernel(text, padding_index, emb, W1, b1, W2, b2, W3, b3):
    text = text.astype(jnp.int32)
    pad = jnp.asarray(padding_index, jnp.int32).reshape(1)
    textp = jnp.concatenate(
        [text[:SC_ROWS],
         jnp.broadcast_to(pad.reshape(1, 1), (SC_ROWS, SEQP - SEQ))], axis=1)
    idx_flat = textp.reshape(SC_ROWS * SEQP)
    sums_sc = _sc_sums(idx_flat, emb).reshape(SC_ROWS, EMB)
    sums_tc = _tc_gather(text[SC_ROWS:], emb)
    sums = jnp.concatenate([sums_sc, sums_tc], axis=0)

    half = BATCH // 2
    args = (W1, b1.reshape(1, HID), W2, b2.reshape(1, HID), W3,
            b3.reshape(1, NCLS))
    la = _mlp(pad, text[:half], sums[:half], *args)
    lb = _mlp(pad, text[half:], sums[half:], *args)
    return jnp.concatenate([la, lb], axis=0)[:, :NCLS]


# speedup vs baseline: 1.0969x; 1.0969x over previous
import jax
import jax.numpy as jnp
from jax import lax
from jax.experimental import pallas as pl
from jax.experimental.pallas import tpu as pltpu
from jax.experimental.pallas import tpu_sc as plsc

VOCAB = 100000
EMB = 128
HID = 1024
NCLS = 1000
BATCH = 4096
SEQ = 50

NC = 2
NS = 16
NW = NC * NS
SEQP = 56
RPC = 2
IDXC = RPC * SEQP
LANES = 16
EV = EMB // LANES

SC_ROWS = 640
TC_ROWS = BATCH - SC_ROWS
BPW = SC_ROWS // NW
NCHUNK = BPW // RPC

NCLS_PAD = 1024
BM = 512
BMG = 576


def _sc_body(idx_hbm, emb_hbm, out_hbm, idx_v, rows0, rows1, out_v, sem0, sem1):
    wid = lax.axis_index("s") * NC + lax.axis_index("c")
    pltpu.sync_copy(idx_hbm.at[pl.ds(wid * (NCHUNK * IDXC), NCHUNK * IDXC)], idx_v)

    bufs = (rows0, rows1)
    sems = (sem0, sem1)

    def fire(c, j):
        pltpu.async_copy(emb_hbm.at[idx_v.at[pl.ds(c * IDXC, IDXC)]], bufs[j], sems[j])

    for j in range(2):
        fire(j, j)

    def pair(p, carry):
        for j in range(2):
            c = 2 * p + j
            pltpu.make_async_copy(emb_hbm.at[idx_v.at[pl.ds(c * IDXC, IDXC)]], bufs[j], sems[j]).wait()
            rows_v = bufs[j]
            for r in range(RPC):
                accs = [rows_v[r * SEQP, pl.ds(e * LANES, LANES)]
                        for e in range(EV)]
                for s in range(1, SEQP):
                    row = r * SEQP + s
                    for e in range(EV):
                        accs[e] = accs[e] + rows_v[row, pl.ds(e * LANES, LANES)]
                base = (c * RPC + r) * EMB
                for e in range(EV):
                    out_v[pl.ds(base + e * LANES, LANES)] = accs[e]

            @pl.when(c + 2 < NCHUNK)
            def _():
                fire(c + 2, j)

        return carry

    lax.fori_loop(0, NCHUNK // 2, pair, 0)
    pltpu.sync_copy(out_v, out_hbm.at[pl.ds(wid * (BPW * EMB), BPW * EMB)])


def _sc_sums(idx_flat, emb):
    mesh = plsc.VectorSubcoreMesh(core_axis_name="c", subcore_axis_name="s")
    return pl.kernel(
        _sc_body,
        out_type=jax.ShapeDtypeStruct((SC_ROWS * EMB,), jnp.float32),
        mesh=mesh,
        scratch_types=[
            pltpu.VMEM((NCHUNK * IDXC,), jnp.int32),
            pltpu.VMEM((IDXC, EMB), jnp.float32),
            pltpu.VMEM((IDXC, EMB), jnp.float32),
            pltpu.VMEM((BPW * EMB,), jnp.float32),
            pltpu.SemaphoreType.DMA,
            pltpu.SemaphoreType.DMA,
        ],
    )(idx_flat, emb)


def _tc_gather_body(text_smem, table_ref, out_ref):
    def row(b, carry):
        accs = [table_ref[pl.ds(text_smem[b, s], 1), :] for s in range(4)]
        for s in range(4, SEQ):
            accs[s % 4] = accs[s % 4] + table_ref[pl.ds(text_smem[b, s], 1), :]
        out_ref[pl.ds(b, 1), :] = (accs[0] + accs[1]) + (accs[2] + accs[3])
        return carry

    lax.fori_loop(0, BMG, row, 0, unroll=32)


def _tc_gather(text_tc, emb):
    return pl.pallas_call(
        _tc_gather_body,
        grid=(TC_ROWS // BMG,),
        in_specs=[
            pl.BlockSpec((BMG, SEQ), lambda i: (i, 0),
                         memory_space=pltpu.SMEM),
            pl.BlockSpec((VOCAB, EMB), lambda i: (0, 0)),
        ],
        out_specs=pl.BlockSpec((BMG, EMB), lambda i: (i, 0)),
        out_shape=jax.ShapeDtypeStruct((TC_ROWS, EMB), jnp.float32),
    )(text_tc, emb)


def _mlp_body(pad_ref, text_ref, sums_ref, w1_ref, b1_ref, w2_ref, b2_ref,
              w3_ref, b3_ref, out_ref):
    cnt = jnp.sum((text_ref[...] != pad_ref[0]).astype(jnp.float32), axis=1,
                  keepdims=True)
    x = sums_ref[...] / cnt
    h = jnp.dot(x, w1_ref[...], preferred_element_type=jnp.float32)
    h = jnp.maximum(h + b1_ref[...], 0.0)
    h = jnp.dot(h, w2_ref[...], preferred_element_type=jnp.float32)
    h = jnp.maximum(h + b2_ref[...], 0.0)
    h = jnp.dot(h, w3_ref[...], preferred_element_type=jnp.float32)
    h = h + b3_ref[...]
    out_ref[...] = jnp.concatenate(
        [h, jnp.zeros((BM, NCLS_PAD - NCLS), jnp.float32)], axis=1)


def _mlp(pad, text, sums, W1, b1, W2, b2, W3p, b3p):
    return pl.pallas_call(
        _mlp_body,
        grid=(BATCH // BM,),
        in_specs=[
            pl.BlockSpec(memory_space=pltpu.SMEM),
            pl.BlockSpec((BM, SEQ), lambda i: (i, 0)),
            pl.BlockSpec((BM, EMB), lambda i: (i, 0)),
            pl.BlockSpec((EMB, HID), lambda i: (0, 0)),
            pl.BlockSpec((1, HID), lambda i: (0, 0)),
            pl.BlockSpec((HID, HID), lambda i: (0, 0)),
            pl.BlockSpec((1, HID), lambda i: (0, 0)),
            pl.BlockSpec((HID, NCLS), lambda i: (0, 0)),
            pl.BlockSpec((1, NCLS), lambda i: (0, 0)),
        ],
        out_specs=pl.BlockSpec((BM, NCLS_PAD), lambda i: (i, 0)),
        out_shape=jax.ShapeDtypeStruct((BATCH, NCLS_PAD), jnp.float32),
    )(pad, text, sums, W1, b1, W2, b2, W3p, b3p)


def kernel(text, padding_index, emb, W1, b1, W2, b2, W3, b3):
    text = text.astype(jnp.int32)
    pad = jnp.asarray(padding_index, jnp.int32).reshape(1)
    textp = jnp.concatenate(
        [text[:SC_ROWS],
         jnp.broadcast_to(pad.reshape(1, 1), (SC_ROWS, SEQP - SEQ))], axis=1)
    idx_flat = textp.reshape(SC_ROWS * SEQP)
    sums_sc = _sc_sums(idx_flat, emb).reshape(SC_ROWS, EMB)
    sums_tc = _tc_gather(text[SC_ROWS:], emb)
    sums = jnp.concatenate([sums_sc, sums_tc], axis=0)

    logits = _mlp(pad, text, sums, W1, b1.reshape(1, HID), W2,
                  b2.reshape(1, HID), W3, b3.reshape(1, NCLS))
    return logits[:, :NCLS]
